# baseline (speedup 1.0000x reference)
import functools

import jax
import jax.numpy as jnp
from jax import lax
from jax.experimental import pallas as pl
from jax.experimental.pallas import tpu as pltpu
from jax.experimental.pallas import tpu_sc as plsc

_NC = 2
_NS = 16
_NW = _NC * _NS
_L = 16


def _tc_body(x_ref, w_ref, o_ref):
    y = x_ref[...] + w_ref[...][None]
    o_ref[0] = y
    o_ref[1] = y


def _kernel_tc(x, embed_weight):
    B, S, D = x.shape
    TS = 512
    out = pl.pallas_call(
        _tc_body,
        grid=(S // TS,),
        in_specs=[
            pl.BlockSpec((B, TS, D), lambda s: (0, s, 0)),
            pl.BlockSpec((TS, D), lambda s: (s, 0)),
        ],
        out_specs=pl.BlockSpec((B, B, TS, D), lambda s: (0, 0, s, 0)),
        out_shape=jax.ShapeDtypeStruct((B, B, S, D), x.dtype),
    )(x, embed_weight)
    return out


def _kernel_sc(x, embed_weight):
    B, S, D = x.shape
    R = B * S
    RPW = R // _NW
    C = 16
    NB = 2
    NCHUNK = RPW // C
    CW = C * D
    UN = 16

    xf = x.reshape(R * D)
    wf = embed_weight.reshape(-1)
    mesh = plsc.VectorSubcoreMesh(core_axis_name="c", subcore_axis_name="s")

    @functools.partial(
        pl.kernel,
        mesh=mesh,
        out_type=jax.ShapeDtypeStruct((2 * R * D,), jnp.float32),
        scratch_types=(
            [pltpu.VMEM_SHARED((_NS * NB * CW,), jnp.float32)]
            + [pltpu.VMEM((CW,), jnp.float32) for _ in range(NB)]
            + [pltpu.SemaphoreType.DMA for _ in range(2 * NB)]
        ),
    )
    def k(x_hbm, w_hbm, out_hbm, *bufs):
        sid = lax.axis_index("s")
        xs = bufs[0]
        xv = [xs.at[pl.ds((sid * NB + b) * CW, CW)] for b in range(NB)]
        wv = bufs[1:1 + NB]
        yv = wv
        sin = bufs[1 + NB:1 + 2 * NB]
        sout = bufs[1 + 2 * NB:1 + 3 * NB]

        wid = lax.axis_index("s") * _NC + lax.axis_index("c")
        base = wid * RPW
        sbase = lax.rem(base, S)

        in_h = [None] * NB
        out_h = [None] * NB

        def start_in(i):
            b = i % NB
            off = base * D + i * CW
            woff = sbase * D + i * CW
            h1 = pltpu.async_copy(x_hbm.at[pl.ds(off, CW)], xv[b], sin[b])
            h2 = pltpu.async_copy(w_hbm.at[pl.ds(woff, CW)], wv[b], sin[b])
            in_h[b] = (h1, h2)

        for i in range(NB):
            start_in(i)

        for i in range(NCHUNK):
            b = i % NB
            for h in in_h[b]:
                h.wait()
            if out_h[b] is not None:
                for h in out_h[b]:
                    h.wait()

            def vstep(t, c2, _b=b):
                for u in range(UN):
                    sl = pl.ds((t * UN + u) * _L, _L)
                    yv[_b][sl] = xv[_b][sl] + wv[_b][sl]
                return c2


            off = base * D + i * CW
            h1 = pltpu.async_copy(xv[b], out_hbm.at[pl.ds(off, CW)], sout[b])
            h2 = pltpu.async_copy(xv[b], out_hbm.at[pl.ds(R * D + off, CW)], sout[b])
            out_h[b] = (h1, h2)
            if i + NB < NCHUNK:
                start_in(i + NB)

        for b in range(NB):
            if out_h[b] is not None:
                for h in out_h[b]:
                    h.wait()

    out = k(xf, wf)
    return out.reshape(B, B, S, D)


def kernel(x, embed_weight):
    return _kernel_sc(x, embed_weight)


# speedup vs baseline: 5.1669x; 5.1669x over previous
import functools

import jax
import jax.numpy as jnp
from jax import lax
from jax.experimental import pallas as pl
from jax.experimental.pallas import tpu as pltpu
from jax.experimental.pallas import tpu_sc as plsc

_NC = 2
_NS = 16
_NW = _NC * _NS
_L = 16


def _tc_body(x_ref, w_ref, o_ref):
    y = x_ref[...] + w_ref[...][None]
    o_ref[0] = y
    o_ref[1] = y


def _kernel_tc(x, embed_weight):
    B, S, D = x.shape
    TS = 512
    DS = 1024
    out = pl.pallas_call(
        _tc_body,
        grid=(S // TS, D // DS),
        in_specs=[
            pl.BlockSpec((B, TS, DS), lambda s, d: (0, s, d)),
            pl.BlockSpec((TS, DS), lambda s, d: (s, d)),
        ],
        out_specs=pl.BlockSpec((B, B, TS, DS), lambda s, d: (0, 0, s, d)),
        out_shape=jax.ShapeDtypeStruct((B, B, S, D), x.dtype),
    )(x, embed_weight)
    return out


def _kernel_sc(x, embed_weight):
    B, S, D = x.shape
    R = B * S
    RPW = R // _NW
    C = 16
    NB = 2
    NCHUNK = RPW // C
    CW = C * D
    UN = 16

    xf = x.reshape(R * D)
    wf = embed_weight.reshape(-1)
    mesh = plsc.VectorSubcoreMesh(core_axis_name="c", subcore_axis_name="s")

    @functools.partial(
        pl.kernel,
        mesh=mesh,
        out_type=jax.ShapeDtypeStruct((2 * R * D,), jnp.float32),
        scratch_types=(
            [pltpu.VMEM_SHARED((_NS * NB * CW,), jnp.float32)]
            + [pltpu.VMEM((CW,), jnp.float32) for _ in range(NB)]
            + [pltpu.SemaphoreType.DMA for _ in range(2 * NB)]
        ),
    )
    def k(x_hbm, w_hbm, out_hbm, *bufs):
        sid = lax.axis_index("s")
        xs = bufs[0]
        xv = [xs.at[pl.ds((sid * NB + b) * CW, CW)] for b in range(NB)]
        wv = bufs[1:1 + NB]
        yv = wv
        sin = bufs[1 + NB:1 + 2 * NB]
        sout = bufs[1 + 2 * NB:1 + 3 * NB]

        wid = lax.axis_index("s") * _NC + lax.axis_index("c")
        base = wid * RPW
        sbase = lax.rem(base, S)

        in_h = [None] * NB
        out_h = [None] * NB

        def start_in(i):
            b = i % NB
            off = base * D + i * CW
            woff = sbase * D + i * CW
            h1 = pltpu.async_copy(x_hbm.at[pl.ds(off, CW)], xv[b], sin[b])
            h2 = pltpu.async_copy(w_hbm.at[pl.ds(woff, CW)], wv[b], sin[b])
            in_h[b] = (h1, h2)

        for i in range(NB):
            start_in(i)

        for i in range(NCHUNK):
            b = i % NB
            for h in in_h[b]:
                h.wait()
            if out_h[b] is not None:
                for h in out_h[b]:
                    h.wait()

            def vstep(t, c2, _b=b):
                for u in range(UN):
                    sl = pl.ds((t * UN + u) * _L, _L)
                    yv[_b][sl] = xv[_b][sl] + wv[_b][sl]
                return c2


            off = base * D + i * CW
            h1 = pltpu.async_copy(xv[b], out_hbm.at[pl.ds(off, CW)], sout[b])
            h2 = pltpu.async_copy(xv[b], out_hbm.at[pl.ds(R * D + off, CW)], sout[b])
            out_h[b] = (h1, h2)
            if i + NB < NCHUNK:
                start_in(i + NB)

        for b in range(NB):
            if out_h[b] is not None:
                for h in out_h[b]:
                    h.wait()

    out = k(xf, wf)
    return out.reshape(B, B, S, D)


def kernel(x, embed_weight):
    return _kernel_tc(x, embed_weight)
